# baseline (speedup 1.0000x reference)
import jax
import jax.numpy as jnp
from jax.experimental import pallas as pl
from jax.experimental.pallas import tpu as pltpu

N = 10000
BR = 400

_P = jax.lax.Precision.DEFAULT


def _pass1_kernel(x_ref, a_ref, w0_ref, w1_ref, g_ref, h0t_ref):
    @pl.when(pl.program_id(0) == 0)
    def _():
        h0t_ref[...] = jax.lax.dot_general(
            w0_ref[...], x_ref[...], (((0,), (1,)), ((), ())),
            preferred_element_type=jnp.float32, precision=_P,
        )

    zt = jax.lax.dot_general(
        h0t_ref[...], a_ref[...], (((1,), (1,)), ((), ())),
        preferred_element_type=jnp.float32, precision=_P,
    )
    zt = jnp.maximum(zt, 0.0)
    g_ref[...] = jax.lax.dot_general(
        w1_ref[...], zt, (((0,), (0,)), ((), ())),
        preferred_element_type=jnp.float32, precision=_P,
    ).T


def _pass2_kernel(a_ref, g_ref, out_ref, gt_ref):
    @pl.when(pl.program_id(0) == 0)
    def _():
        gt_ref[...] = g_ref[...].T

    lt = jax.lax.dot_general(
        gt_ref[...], a_ref[...], (((1,), (1,)), ((), ())),
        preferred_element_type=jnp.float32, precision=_P,
    )
    m = jnp.max(lt, axis=0, keepdims=True)
    e = jnp.exp(lt - m)
    out_ref[...] = (e / jnp.sum(e, axis=0, keepdims=True)).T


def kernel(x, a, W0, W1):
    n, f_in = x.shape
    c0 = W0.shape[1]
    c1 = W1.shape[1]
    nb = n // BR
    grid = (nb,)

    g = pl.pallas_call(
        _pass1_kernel,
        grid=grid,
        in_specs=[
            pl.BlockSpec((n, f_in), lambda i: (0, 0)),
            pl.BlockSpec((BR, n), lambda i: (i, 0)),
            pl.BlockSpec((f_in, c0), lambda i: (0, 0)),
            pl.BlockSpec((c0, c1), lambda i: (0, 0)),
        ],
        out_specs=pl.BlockSpec((BR, c1), lambda i: (i, 0)),
        out_shape=jax.ShapeDtypeStruct((n, c1), jnp.float32),
        scratch_shapes=[pltpu.VMEM((c0, n), jnp.float32)],
    )(x, a, W0, W1)

    out = pl.pallas_call(
        _pass2_kernel,
        grid=grid,
        in_specs=[
            pl.BlockSpec((BR, n), lambda i: (i, 0)),
            pl.BlockSpec((n, c1), lambda i: (0, 0)),
        ],
        out_specs=pl.BlockSpec((BR, c1), lambda i: (i, 0)),
        out_shape=jax.ShapeDtypeStruct((n, c1), jnp.float32),
        scratch_shapes=[pltpu.VMEM((c1, n), jnp.float32)],
    )(a, g)
    return out


# speedup vs baseline: 1.0067x; 1.0067x over previous
import jax
import jax.numpy as jnp
from jax.experimental import pallas as pl
from jax.experimental.pallas import tpu as pltpu

N = 10000
BR = 400

_P = jax.lax.Precision.DEFAULT


def _h0_kernel(x_ref, w0_ref, h0t_ref):
    h0t_ref[...] = jax.lax.dot_general(
        w0_ref[...], x_ref[...], (((0,), (1,)), ((), ())),
        preferred_element_type=jnp.float32, precision=_P,
    )


def _pass1_kernel(a_ref, h0t_ref, w1_ref, gt3_ref):
    zt = jax.lax.dot_general(
        h0t_ref[...], a_ref[...], (((1,), (1,)), ((), ())),
        preferred_element_type=jnp.float32, precision=_P,
    )
    zt = jnp.maximum(zt, 0.0)
    gt3_ref[...] = jax.lax.dot_general(
        w1_ref[...], zt, (((0,), (0,)), ((), ())),
        preferred_element_type=jnp.float32, precision=_P,
    )[None]


def _pass2_kernel(a_ref, gt3_ref, out_ref, gt_ref):
    @pl.when(pl.program_id(0) == 0)
    def _():
        nb = gt3_ref.shape[0]
        gt_ref[...] = jnp.concatenate(
            [gt3_ref[b] for b in range(nb)], axis=-1)

    lt = jax.lax.dot_general(
        gt_ref[...], a_ref[...], (((1,), (1,)), ((), ())),
        preferred_element_type=jnp.float32, precision=_P,
    )
    m = jnp.max(lt, axis=0, keepdims=True)
    e = jnp.exp(lt - m)
    out_ref[...] = (e / jnp.sum(e, axis=0, keepdims=True)).T


def kernel(x, a, W0, W1):
    n, f_in = x.shape
    c0 = W0.shape[1]
    c1 = W1.shape[1]
    nb = n // BR
    grid = (nb,)

    h0t = pl.pallas_call(
        _h0_kernel,
        out_shape=jax.ShapeDtypeStruct((c0, n), jnp.float32),
    )(x, W0)

    gt3 = pl.pallas_call(
        _pass1_kernel,
        grid=grid,
        in_specs=[
            pl.BlockSpec((BR, n), lambda i: (i, 0)),
            pl.BlockSpec((c0, n), lambda i: (0, 0)),
            pl.BlockSpec((c0, c1), lambda i: (0, 0)),
        ],
        out_specs=pl.BlockSpec((1, c1, BR), lambda i: (i, 0, 0)),
        out_shape=jax.ShapeDtypeStruct((nb, c1, BR), jnp.float32),
    )(a, h0t, W1)

    out = pl.pallas_call(
        _pass2_kernel,
        grid=grid,
        in_specs=[
            pl.BlockSpec((BR, n), lambda i: (i, 0)),
            pl.BlockSpec((nb, c1, BR), lambda i: (0, 0, 0)),
        ],
        out_specs=pl.BlockSpec((BR, c1), lambda i: (i, 0)),
        out_shape=jax.ShapeDtypeStruct((n, c1), jnp.float32),
        scratch_shapes=[pltpu.VMEM((c1, n), jnp.float32)],
    )(a, gt3)
    return out


# speedup vs baseline: 1.0192x; 1.0124x over previous
import jax
import jax.numpy as jnp
from jax.experimental import pallas as pl
from jax.experimental.pallas import tpu as pltpu

N = 10000
BR = 400

_P = jax.lax.Precision.DEFAULT


def _h0_kernel(x_ref, w0_ref, h0t_ref):
    h0t_ref[...] = jax.lax.dot_general(
        w0_ref[...], x_ref[...], (((0,), (1,)), ((), ())),
        preferred_element_type=jnp.float32, precision=_P,
    )


def _gcn_kernel(a_ref, h0t_ref, w1_ref, out_ref, gt3_ref, gt_ref):
    i = pl.program_id(0)
    nb = gt3_ref.shape[0]

    @pl.when(i < nb)
    def _():
        zt = jax.lax.dot_general(
            h0t_ref[...], a_ref[...], (((1,), (1,)), ((), ())),
            preferred_element_type=jnp.float32, precision=_P,
        )
        zt = jnp.maximum(zt, 0.0)
        gt3_ref[i] = jax.lax.dot_general(
            w1_ref[...], zt, (((0,), (0,)), ((), ())),
            preferred_element_type=jnp.float32, precision=_P,
        )

    @pl.when(i == nb)
    def _():
        gt_ref[...] = jnp.concatenate(
            [gt3_ref[b] for b in range(nb)], axis=-1)

    @pl.when(i >= nb)
    def _():
        lt = jax.lax.dot_general(
            gt_ref[...], a_ref[...], (((1,), (1,)), ((), ())),
            preferred_element_type=jnp.float32, precision=_P,
        )
        m = jnp.max(lt, axis=0, keepdims=True)
        e = jnp.exp(lt - m)
        out_ref[...] = (e / jnp.sum(e, axis=0, keepdims=True)).T


def kernel(x, a, W0, W1):
    n, f_in = x.shape
    c0 = W0.shape[1]
    c1 = W1.shape[1]
    nb = n // BR

    h0t = pl.pallas_call(
        _h0_kernel,
        out_shape=jax.ShapeDtypeStruct((c0, n), jnp.float32),
    )(x, W0)

    out = pl.pallas_call(
        _gcn_kernel,
        grid=(2 * nb,),
        in_specs=[
            pl.BlockSpec((BR, n), lambda i: (i % (n // BR), 0)),
            pl.BlockSpec((c0, n), lambda i: (0, 0)),
            pl.BlockSpec((c0, c1), lambda i: (0, 0)),
        ],
        out_specs=pl.BlockSpec((BR, c1), lambda i: (i % (n // BR), 0)),
        out_shape=jax.ShapeDtypeStruct((n, c1), jnp.float32),
        scratch_shapes=[
            pltpu.VMEM((nb, c1, BR), jnp.float32),
            pltpu.VMEM((c1, n), jnp.float32),
        ],
    )(a, h0t, W1)
    return out


# speedup vs baseline: 1.0267x; 1.0073x over previous
import jax
import jax.numpy as jnp
from jax.experimental import pallas as pl
from jax.experimental.pallas import tpu as pltpu

N = 10000
BR = 400

_P = jax.lax.Precision.DEFAULT


def _h0_kernel(x_ref, w0_ref, h0t_ref):
    h0t_ref[...] = jax.lax.dot_general(
        w0_ref[...], x_ref[...], (((0,), (1,)), ((), ())),
        preferred_element_type=jnp.float32, precision=_P,
    )


def _gcn_kernel(a_ref, h0t_ref, w1_ref, out_ref, gt3_ref, gt_ref):
    i = pl.program_id(0)
    nb = gt3_ref.shape[0]

    @pl.when(i < nb)
    def _():
        zt = jax.lax.dot_general(
            h0t_ref[...], a_ref[...], (((1,), (1,)), ((), ())),
            preferred_element_type=jnp.float32, precision=_P,
        )
        zt = jnp.maximum(zt, 0.0)
        gt3_ref[i] = jax.lax.dot_general(
            w1_ref[...], zt, (((0,), (0,)), ((), ())),
            preferred_element_type=jnp.float32, precision=_P,
        )

    @pl.when(i == nb)
    def _():
        gt_ref[...] = jnp.concatenate(
            [gt3_ref[b] for b in range(nb)], axis=-1)

    @pl.when(i >= nb)
    def _():
        lt = jax.lax.dot_general(
            gt_ref[...], a_ref[...], (((1,), (1,)), ((), ())),
            preferred_element_type=jnp.float32, precision=_P,
        )
        m = jnp.max(lt, axis=0, keepdims=True)
        e = jnp.exp(lt - m)
        out_ref[...] = (e / jnp.sum(e, axis=0, keepdims=True)).T


def kernel(x, a, W0, W1):
    n, f_in = x.shape
    c0 = W0.shape[1]
    c1 = W1.shape[1]
    nb = n // BR

    h0t = pl.pallas_call(
        _h0_kernel,
        out_shape=jax.ShapeDtypeStruct((c0, n), jnp.float32),
    )(x, W0)

    out = pl.pallas_call(
        _gcn_kernel,
        grid=(2 * nb,),
        in_specs=[
            pl.BlockSpec((BR, n), lambda i: (i % (n // BR), 0)),
            pl.BlockSpec((c0, n), lambda i: (0, 0)),
            pl.BlockSpec((c0, c1), lambda i: (0, 0)),
        ],
        out_specs=pl.BlockSpec(
            (BR, c1), lambda i: (jnp.maximum(i - n // BR, 0), 0)),
        out_shape=jax.ShapeDtypeStruct((n, c1), jnp.float32),
        scratch_shapes=[
            pltpu.VMEM((nb, c1, BR), jnp.float32),
            pltpu.VMEM((c1, n), jnp.float32),
        ],
    )(a, h0t, W1)
    return out
